# baseline (speedup 1.0000x reference)
import functools
import math

import jax
import jax.numpy as jnp
from jax import lax
from jax.experimental import pallas as pl
from jax.experimental.pallas import tpu as pltpu
from jax.experimental.pallas import tpu_sc as plsc

N = 10000
E = 160000
H = 64
F = 256
G = 64

NC = 2
NS = 16
NW = NC * NS

_LOG2 = math.log(2.0)

TE = 3200
TN = 1000


def _ssp(v):
    return jnp.maximum(v, 0.0) + jnp.log1p(jnp.exp(-jnp.abs(v))) - _LOG2


def _sc_gather(table, idxs):
    K = len(idxs)
    M = idxs[0].shape[0]
    D = table.shape[1]
    pw_exact = M // NW
    pad = max((w * pw_exact) % 8 for w in range(NW))
    pw = pw_exact + pad
    for w in range(NW):
        assert w * pw_exact - (w * pw_exact) % 8 + pw <= M
    CH = 128
    nfull, tail = divmod(pw, CH)
    mesh = plsc.VectorSubcoreMesh(core_axis_name="c", subcore_axis_name="s")

    def body(table_hbm, *refs):
        idx_hbms = refs[:K]
        out_hbm = refs[K]
        sc = refs[K + 1:]
        idx_vs = sc[:K]
        rbufs = sc[K:K + 2 * K]
        rtails = sc[3 * K:3 * K + K]
        gsems = sc[4 * K:4 * K + 2 * K]
        wsems = sc[6 * K:6 * K + 2 * K]
        stail = sc[8 * K]
        wid = lax.axis_index("s") * NC + lax.axis_index("c")
        b = wid * pw_exact
        base0 = pl.multiple_of(b - lax.rem(b, 8), 8)
        for k in range(K):
            pltpu.sync_copy(idx_hbms[k].at[pl.ds(base0, pw)], idx_vs[k])
        gds, wds = {}, {}
        for j in range(nfull):
            b = j % 2
            for k in range(K):
                if j >= 2:
                    wds[(j - 2, k)].wait()
                gds[(j, k)] = pltpu.async_copy(
                    table_hbm.at[idx_vs[k].at[pl.ds(j * CH, CH)]],
                    rbufs[2 * k + b], gsems[2 * k + b])
            if j >= 1:
                pb = (j - 1) % 2
                for k in range(K):
                    gds[(j - 1, k)].wait()
                    wds[(j - 1, k)] = pltpu.async_copy(
                        rbufs[2 * k + pb],
                        out_hbm.at[pl.ds(base0 + (j - 1) * CH, CH),
                                   pl.ds(64 * k, D)],
                        wsems[2 * k + pb])
        if nfull >= 1:
            j = nfull - 1
            b = j % 2
            for k in range(K):
                gds[(j, k)].wait()
                wds[(j, k)] = pltpu.async_copy(
                    rbufs[2 * k + b],
                    out_hbm.at[pl.ds(base0 + j * CH, CH), pl.ds(64 * k, D)],
                    wsems[2 * k + b])
        for j in (nfull - 2, nfull - 1):
            if j >= 0:
                for k in range(K):
                    wds[(j, k)].wait()
        if tail:
            base = base0 + nfull * CH
            for k in range(K):
                pltpu.async_copy(
                    table_hbm.at[idx_vs[k].at[pl.ds(nfull * CH, tail)]],
                    rtails[k], stail).wait()
                pltpu.async_copy(
                    rtails[k],
                    out_hbm.at[pl.ds(base, tail), pl.ds(64 * k, D)],
                    stail).wait()

    scratch = (
        [pltpu.VMEM((pw,), jnp.int32)] * K
        + [pltpu.VMEM((CH, D), jnp.float32)] * (2 * K)
        + [pltpu.VMEM((max(tail, 8), D), jnp.float32)] * K
        + [pltpu.SemaphoreType.DMA] * (4 * K + 1)
    )
    k = pl.kernel(
        body,
        out_type=jax.ShapeDtypeStruct((M, 128), jnp.float32),
        mesh=mesh,
        scratch_types=scratch,
        compiler_params=pltpu.CompilerParams(use_tc_tiling_on_sc=False),
    )
    return k(table, *idxs)


def _sc_scatter_add(m2, dst, zrows):
    EH = dst.shape[0]
    CH = 125
    nch = EH // (NS * CH)
    nps = N // NS
    mesh = plsc.VectorSubcoreMesh(core_axis_name="c", subcore_axis_name="s")

    def body(m_hbm, dst_hbm, z_hbm, agg_hbm, didx_v, r0, r1, agg_s,
             m0, m1, s0, s1):
        c = lax.axis_index("c")
        s = lax.axis_index("s")
        pltpu.sync_copy(z_hbm, agg_s.at[pl.ds(s * nps, nps)])
        pltpu.sync_copy(dst_hbm.at[pl.ds(s * nch, nch)], didx_v)
        plsc.subcore_barrier()
        bufs = [(r0, m0, s0), (r1, m1, s1)]
        mds, sds = {}, {}
        for j in range(nch):
            rows, msem, ssem = bufs[j % 2]
            if j >= 2:
                sds[j - 2].wait()
            mds[j] = pltpu.async_copy(
                m_hbm.at[pl.ds((s * nch + j) * CH, CH),
                         pl.ds(pl.multiple_of(c * 128, 128), 128)],
                rows, msem)
            if j >= 1:
                prows, _, pssem = bufs[(j - 1) % 2]
                mds[j - 1].wait()
                sds[j - 1] = pltpu.async_copy(
                    prows, agg_s.at[didx_v.at[j - 1]], pssem, add=True)
        j = nch - 1
        rows, _, ssem = bufs[j % 2]
        mds[j].wait()
        sds[j] = pltpu.async_copy(rows, agg_s.at[didx_v.at[j]], ssem, add=True)
        sds[nch - 2].wait()
        sds[nch - 1].wait()
        plsc.subcore_barrier()
        pltpu.sync_copy(agg_s.at[pl.ds(s * nps, nps)],
                        agg_hbm.at[c, pl.ds(s * nps, nps), :])

    k = pl.kernel(
        body,
        out_type=jax.ShapeDtypeStruct((2, N, 128), jnp.float32),
        mesh=mesh,
        scratch_types=[
            pltpu.VMEM((nch, CH), jnp.int32),
            pltpu.VMEM((CH, 128), jnp.float32),
            pltpu.VMEM((CH, 128), jnp.float32),
            pltpu.VMEM_SHARED((N, 128), jnp.float32),
            pltpu.SemaphoreType.DMA,
            pltpu.SemaphoreType.DMA,
            pltpu.SemaphoreType.DMA,
            pltpu.SemaphoreType.DMA,
        ],
        compiler_params=pltpu.CompilerParams(use_tc_tiling_on_sc=False),
    )
    return k(m2, dst.reshape(EH // CH, CH), zrows)


def _tc_node_init(vrows, pos, pos_emb_w):
    def body(v_ref, p_ref, w_ref, h_ref):
        pe = jnp.dot(p_ref[...], w_ref[...], preferred_element_type=jnp.float32)
        h_ref[...] = jnp.concatenate([v_ref[:, :16], pe], axis=-1)

    return pl.pallas_call(
        body,
        grid=(N // TN,),
        in_specs=[
            pl.BlockSpec((TN, 128), lambda i: (i, 0)),
            pl.BlockSpec((TN, 3), lambda i: (i, 0)),
            pl.BlockSpec((3, 48), lambda i: (0, 0)),
        ],
        out_specs=pl.BlockSpec((TN, H), lambda i: (i, 0)),
        out_shape=jax.ShapeDtypeStruct((N, H), jnp.float32),
    )(vrows, pos, pos_emb_w)


def _tc_edge(l, ea_prev, gsd, weights):
    want_ea = l < 2
    EH = gsd.shape[0]
    grid = (EH // TE,)

    def body(*refs):
        if l == 0:
            (eat_ref, gs_ref, eew_ref, fw1_ref, fb1_ref, fw2_ref, fb2_ref,
             lin1_ref, m_ref, *rest) = refs
            ea = jnp.dot(eat_ref[...], eew_ref[...],
                         preferred_element_type=jnp.float32)
            gs = gs_ref[:, :H]
        else:
            (eap_ref, gsd_ref, emw_ref, emb_ref, fw1_ref, fb1_ref,
             fw2_ref, fb2_ref, lin1_ref, m_ref, *rest) = refs
            eap = eap_ref[...]
            gs = gsd_ref[:, :H]
            hs = gs + gsd_ref[:, H:]
            e_in = jnp.concatenate(
                [jnp.maximum(eap, 0.0), jnp.maximum(hs, 0.0)], axis=-1)
            ea = jnp.tanh(
                jnp.dot(e_in, emw_ref[...],
                        preferred_element_type=jnp.float32) + emb_ref[...]
            ) + eap
        t = _ssp(jnp.dot(ea, fw1_ref[...],
                         preferred_element_type=jnp.float32) + fb1_ref[...])
        W = jnp.dot(t, fw2_ref[...],
                    preferred_element_type=jnp.float32) + fb2_ref[...]
        m = jnp.dot(gs, lin1_ref[...],
                    preferred_element_type=jnp.float32) * W
        m_ref[...] = m
        if want_ea:
            rest[0][...] = ea

    wspec = lambda shape: pl.BlockSpec(shape, lambda i: tuple(0 for _ in shape))
    if l == 0:
        edge_attr, eew = ea_prev, weights["eew"]
        boff = weights["eoff"] // TE
        in_arrays = [edge_attr, gsd, eew,
                     weights["fw1"], weights["fb1"], weights["fw2"],
                     weights["fb2"], weights["lin1"]]
        in_specs = [
            pl.BlockSpec((TE, 4), lambda i, _b=boff: (i + _b, 0)),
            pl.BlockSpec((TE, 128), lambda i: (i, 0)),
            wspec((4, H)), wspec((H, F)), wspec((1, F)), wspec((F, F)),
            wspec((1, F)), wspec((H, F)),
        ]
    else:
        in_arrays = [ea_prev, gsd,
                     weights["emw"], weights["emb_b"],
                     weights["fw1"], weights["fb1"], weights["fw2"],
                     weights["fb2"], weights["lin1"]]
        in_specs = [
            pl.BlockSpec((TE, H), lambda i: (i, 0)),
            pl.BlockSpec((TE, 128), lambda i: (i, 0)),
            wspec((2 * H, H)), wspec((1, H)),
            wspec((H, F)), wspec((1, F)), wspec((F, F)), wspec((1, F)),
            wspec((H, F)),
        ]
    out_shapes = [jax.ShapeDtypeStruct((EH, 256), jnp.float32)]
    out_specs = [pl.BlockSpec((TE, 256), lambda i: (i, 0))]
    if want_ea:
        out_shapes.append(jax.ShapeDtypeStruct((EH, H), jnp.float32))
        out_specs.append(pl.BlockSpec((TE, H), lambda i: (i, 0)))

    res = pl.pallas_call(
        body, grid=grid, in_specs=in_specs, out_specs=out_specs,
        out_shape=out_shapes,
    )(*in_arrays)
    return res if want_ea else (res[0], None)


def _tc_node(agg2a, agg2b, h_old, lin2w, lin2b, linw, linb):
    def body(aa_ref, ab_ref, h_ref, w1_ref, b1_ref, w2_ref, b2_ref, o_ref):
        a = jnp.concatenate([aa_ref[0] + ab_ref[0], aa_ref[1] + ab_ref[1]],
                            axis=-1)
        t = _ssp(jnp.dot(a, w1_ref[...],
                         preferred_element_type=jnp.float32) + b1_ref[...])
        o = jnp.dot(t, w2_ref[...],
                    preferred_element_type=jnp.float32) + b2_ref[...]
        o_ref[...] = jnp.maximum(o, 0.0) + h_ref[...]

    wspec = lambda shape: pl.BlockSpec(shape, lambda i: tuple(0 for _ in shape))
    return pl.pallas_call(
        body,
        grid=(N // TN,),
        in_specs=[
            pl.BlockSpec((2, TN, 128), lambda i: (0, i, 0)),
            pl.BlockSpec((2, TN, 128), lambda i: (0, i, 0)),
            pl.BlockSpec((TN, H), lambda i: (i, 0)),
            wspec((F, H)), wspec((1, H)), wspec((H, H)), wspec((1, H)),
        ],
        out_specs=pl.BlockSpec((TN, H), lambda i: (i, 0)),
        out_shape=jax.ShapeDtypeStruct((N, H), jnp.float32),
    )(agg2a, agg2b, h_old, lin2w, lin2b, linw, linb)


def _tc_pool_head(h, batchf, head_w1, head_b1, head_w2, head_b2):
    nb = N // TN

    def body(h_ref, b_ref, w1_ref, b1_ref, w2_ref, b2_ref, o_ref, p_acc, c_acc):
        i = pl.program_id(0)

        @pl.when(i == 0)
        def _():
            p_acc[...] = jnp.zeros_like(p_acc)
            c_acc[...] = jnp.zeros_like(c_acc)

        gids = lax.broadcasted_iota(jnp.int32, (1, G), 1).astype(jnp.float32)
        oh = jnp.where(b_ref[...] == gids, 1.0, 0.0)
        p_acc[...] += lax.dot_general(
            oh, h_ref[...], (((0,), (0,)), ((), ())),
            preferred_element_type=jnp.float32)
        c_acc[...] += lax.dot_general(
            oh, jnp.ones((TN, H), jnp.float32), (((0,), (0,)), ((), ())),
            preferred_element_type=jnp.float32)

        @pl.when(i == nb - 1)
        def _():
            gmean = p_acc[...] / jnp.maximum(c_acc[...], 1.0)
            hh = jax.nn.gelu(
                jnp.dot(gmean, w1_ref[...],
                        preferred_element_type=jnp.float32) + b1_ref[...])
            o_ref[...] = jnp.dot(hh, w2_ref[...],
                                 preferred_element_type=jnp.float32) + b2_ref[...]

    wspec = lambda shape: pl.BlockSpec(shape, lambda i: tuple(0 for _ in shape))
    return pl.pallas_call(
        body,
        grid=(nb,),
        in_specs=[
            pl.BlockSpec((TN, H), lambda i: (i, 0)),
            pl.BlockSpec((TN, 1), lambda i: (i, 0)),
            wspec((H, 512)), wspec((1, 512)), wspec((512, 1)), wspec((1, 1)),
        ],
        out_specs=pl.BlockSpec((G, 1), lambda i: (0, 0)),
        out_shape=jax.ShapeDtypeStruct((G, 1), jnp.float32),
        scratch_shapes=[
            pltpu.VMEM((G, H), jnp.float32),
            pltpu.VMEM((G, H), jnp.float32),
        ],
    )(h, batchf, head_w1, head_b1, head_w2, head_b2)


def kernel(x, edge_index, edge_attr, batch, pos, vert_emb, pos_emb_w,
           edge_emb_w, fw1, fb1, fw2, fb2, lin1, lin2w, lin2b, linw, linb,
           emw, emb_b, head_w1, head_b1, head_w2, head_b2):
    x = x.astype(jnp.int32)
    src = edge_index[0].astype(jnp.int32)
    dst = edge_index[1].astype(jnp.int32)
    E2 = E // 2
    srcs = (src[:E2], src[E2:])
    dsts = (dst[:E2], dst[E2:])
    zrows = jnp.zeros((N // NS, 128), jnp.float32)
    batchf = batch.astype(jnp.float32).reshape(N, 1)

    npad = 10240
    xpad = jnp.concatenate([x, jnp.full((npad - N,), 300, jnp.int32)])
    vrows = _sc_gather(vert_emb, [xpad])
    h = _tc_node_init(vrows, pos, pos_emb_w)

    ea = (edge_attr, edge_attr)
    gsd = (_sc_gather(h, [srcs[0]]), _sc_gather(h, [srcs[1]]))
    for l in range(3):
        weights = dict(eew=edge_emb_w, fw1=fw1[l], fb1=fb1[l].reshape(1, F),
                       fw2=fw2[l], fb2=fb2[l].reshape(1, F), lin1=lin1[l])
        if l > 0:
            weights["emw"] = emw[l - 1]
            weights["emb_b"] = emb_b[l - 1].reshape(1, H)
        m2a, eaa = _tc_edge(l, ea[0], gsd[0], dict(weights, eoff=0))
        agg2a = _sc_scatter_add(m2a, dsts[0], zrows)
        m2b, eab = _tc_edge(l, ea[1], gsd[1], dict(weights, eoff=E2))
        agg2b = _sc_scatter_add(m2b, dsts[1], zrows)
        ea = (eaa, eab)
        h = _tc_node(agg2a, agg2b, h, lin2w[l], lin2b[l].reshape(1, H),
                     linw[l], linb[l].reshape(1, H))
        if l < 2:
            gsd = (_sc_gather(h, [srcs[0], dsts[0]]),
                   _sc_gather(h, [srcs[1], dsts[1]]))

    return _tc_pool_head(h, batchf, head_w1, head_b1.reshape(1, 512),
                         head_w2, head_b2.reshape(1, 1))


# speedup vs baseline: 1.4236x; 1.4236x over previous
import functools
import math

import jax
import jax.numpy as jnp
from jax import lax
from jax.experimental import pallas as pl
from jax.experimental.pallas import tpu as pltpu
from jax.experimental.pallas import tpu_sc as plsc

N = 10000
E = 160000
H = 64
F = 256
G = 64

NC = 2
NS = 16
NW = NC * NS

_LOG2 = math.log(2.0)

TE = 3200
TN = 1000


def _ssp(v):
    return jnp.maximum(v, 0.0) + jnp.log1p(jnp.exp(-jnp.abs(v))) - _LOG2


def _sc_gather(table, idxs):
    K = len(idxs)
    M = idxs[0].shape[0]
    D = table.shape[1]
    pw_exact = M // NW
    pad = max((w * pw_exact) % 8 for w in range(NW))
    pw = pw_exact + pad
    for w in range(NW):
        assert w * pw_exact - (w * pw_exact) % 8 + pw <= M
    CH = 128
    nfull, tail = divmod(pw, CH)
    mesh = plsc.VectorSubcoreMesh(core_axis_name="c", subcore_axis_name="s")

    def body(table_hbm, *refs):
        idx_hbms = refs[:K]
        out_hbm = refs[K]
        sc = refs[K + 1:]
        idx_vs = sc[:K]
        rbufs = sc[K:K + 2 * K]
        rtails = sc[3 * K:3 * K + K]
        gsems = sc[4 * K:4 * K + 2 * K]
        wsems = sc[6 * K:6 * K + 2 * K]
        stail = sc[8 * K]
        wid = lax.axis_index("s") * NC + lax.axis_index("c")
        b = wid * pw_exact
        base0 = pl.multiple_of(b - lax.rem(b, 8), 8)
        for k in range(K):
            pltpu.sync_copy(idx_hbms[k].at[pl.ds(base0, pw)], idx_vs[k])
        gds, wds = {}, {}
        for j in range(nfull):
            b = j % 2
            for k in range(K):
                if j >= 2:
                    wds[(j - 2, k)].wait()
                gds[(j, k)] = pltpu.async_copy(
                    table_hbm.at[idx_vs[k].at[pl.ds(j * CH, CH)]],
                    rbufs[2 * k + b], gsems[2 * k + b])
            if j >= 1:
                pb = (j - 1) % 2
                for k in range(K):
                    gds[(j - 1, k)].wait()
                    wds[(j - 1, k)] = pltpu.async_copy(
                        rbufs[2 * k + pb],
                        out_hbm.at[pl.ds(base0 + (j - 1) * CH, CH),
                                   pl.ds(64 * k, D)],
                        wsems[2 * k + pb])
        if nfull >= 1:
            j = nfull - 1
            b = j % 2
            for k in range(K):
                gds[(j, k)].wait()
                wds[(j, k)] = pltpu.async_copy(
                    rbufs[2 * k + b],
                    out_hbm.at[pl.ds(base0 + j * CH, CH), pl.ds(64 * k, D)],
                    wsems[2 * k + b])
        for j in (nfull - 2, nfull - 1):
            if j >= 0:
                for k in range(K):
                    wds[(j, k)].wait()
        if tail:
            base = base0 + nfull * CH
            for k in range(K):
                pltpu.async_copy(
                    table_hbm.at[idx_vs[k].at[pl.ds(nfull * CH, tail)]],
                    rtails[k], stail).wait()
                pltpu.async_copy(
                    rtails[k],
                    out_hbm.at[pl.ds(base, tail), pl.ds(64 * k, D)],
                    stail).wait()

    scratch = (
        [pltpu.VMEM((pw,), jnp.int32)] * K
        + [pltpu.VMEM((CH, D), jnp.float32)] * (2 * K)
        + [pltpu.VMEM((max(tail, 8), D), jnp.float32)] * K
        + [pltpu.SemaphoreType.DMA] * (4 * K + 1)
    )
    k = pl.kernel(
        body,
        out_type=jax.ShapeDtypeStruct((M, 128), jnp.float32),
        mesh=mesh,
        scratch_types=scratch,
        compiler_params=pltpu.CompilerParams(use_tc_tiling_on_sc=False),
    )
    return k(table, *idxs)


def _sc_scatter_add(m2, dst, zrows):
    EH = dst.shape[0]
    CH = 125
    nch = EH // (NS * CH)
    nps = N // NS
    mesh = plsc.VectorSubcoreMesh(core_axis_name="c", subcore_axis_name="s")

    def body(m_hbm, dst_hbm, z_hbm, agg_hbm, didx_v, r0, r1, agg_s,
             m0, m1, s0, s1):
        c = lax.axis_index("c")
        s = lax.axis_index("s")
        pltpu.sync_copy(z_hbm, agg_s.at[pl.ds(s * nps, nps)])
        pltpu.sync_copy(dst_hbm.at[pl.ds(s * nch, nch)], didx_v)
        plsc.subcore_barrier()
        bufs = [(r0, m0, s0), (r1, m1, s1)]
        mds, sds = {}, {}
        for j in range(nch):
            rows, msem, ssem = bufs[j % 2]
            if j >= 2:
                sds[j - 2].wait()
            mds[j] = pltpu.async_copy(
                m_hbm.at[c, pl.ds((s * nch + j) * CH, CH), :], rows, msem)
            if j >= 1:
                prows, _, pssem = bufs[(j - 1) % 2]
                mds[j - 1].wait()
                sds[j - 1] = pltpu.async_copy(
                    prows, agg_s.at[didx_v.at[j - 1]], pssem, add=True)
        j = nch - 1
        rows, _, ssem = bufs[j % 2]
        mds[j].wait()
        sds[j] = pltpu.async_copy(rows, agg_s.at[didx_v.at[j]], ssem, add=True)
        sds[nch - 2].wait()
        sds[nch - 1].wait()
        plsc.subcore_barrier()
        pltpu.sync_copy(agg_s.at[pl.ds(s * nps, nps)],
                        agg_hbm.at[c, pl.ds(s * nps, nps), :])

    k = pl.kernel(
        body,
        out_type=jax.ShapeDtypeStruct((2, N, 128), jnp.float32),
        mesh=mesh,
        scratch_types=[
            pltpu.VMEM((nch, CH), jnp.int32),
            pltpu.VMEM((CH, 128), jnp.float32),
            pltpu.VMEM((CH, 128), jnp.float32),
            pltpu.VMEM_SHARED((N, 128), jnp.float32),
            pltpu.SemaphoreType.DMA,
            pltpu.SemaphoreType.DMA,
            pltpu.SemaphoreType.DMA,
            pltpu.SemaphoreType.DMA,
        ],
        compiler_params=pltpu.CompilerParams(use_tc_tiling_on_sc=False),
    )
    return k(m2, dst.reshape(EH // CH, CH), zrows)


def _tc_node_init(vrows, pos, pos_emb_w):
    def body(v_ref, p_ref, w_ref, h_ref):
        pe = jnp.dot(p_ref[...], w_ref[...], preferred_element_type=jnp.float32)
        h_ref[...] = jnp.concatenate([v_ref[:, :16], pe], axis=-1)

    return pl.pallas_call(
        body,
        grid=(N // TN,),
        in_specs=[
            pl.BlockSpec((TN, 128), lambda i: (i, 0)),
            pl.BlockSpec((TN, 3), lambda i: (i, 0)),
            pl.BlockSpec((3, 48), lambda i: (0, 0)),
        ],
        out_specs=pl.BlockSpec((TN, H), lambda i: (i, 0)),
        out_shape=jax.ShapeDtypeStruct((N, H), jnp.float32),
    )(vrows, pos, pos_emb_w)


def _tc_edge(l, ea_prev, gsd, weights):
    want_ea = l < 2
    EH = gsd.shape[0]
    grid = (EH // TE,)

    def body(*refs):
        if l == 0:
            (eat_ref, gs_ref, eew_ref, fw1_ref, fb1_ref, fw2_ref, fb2_ref,
             lin1_ref, m_ref, *rest) = refs
            ea = jnp.dot(eat_ref[...], eew_ref[...],
                         preferred_element_type=jnp.float32)
            gs = gs_ref[:, :H]
        else:
            (eap_ref, gsd_ref, emw_ref, emb_ref, fw1_ref, fb1_ref,
             fw2_ref, fb2_ref, lin1_ref, m_ref, *rest) = refs
            eap = eap_ref[...]
            gs = gsd_ref[:, :H]
            hs = gs + gsd_ref[:, H:]
            e_in = jnp.concatenate(
                [jnp.maximum(eap, 0.0), jnp.maximum(hs, 0.0)], axis=-1)
            ea = jnp.tanh(
                jnp.dot(e_in, emw_ref[...],
                        preferred_element_type=jnp.float32) + emb_ref[...]
            ) + eap
        t = _ssp(jnp.dot(ea, fw1_ref[...],
                         preferred_element_type=jnp.float32) + fb1_ref[...])
        W = jnp.dot(t, fw2_ref[...],
                    preferred_element_type=jnp.float32) + fb2_ref[...]
        m = jnp.dot(gs, lin1_ref[...],
                    preferred_element_type=jnp.float32) * W
        m_ref[0] = m[:, :128]
        m_ref[1] = m[:, 128:]
        if want_ea:
            rest[0][...] = ea

    wspec = lambda shape: pl.BlockSpec(shape, lambda i: tuple(0 for _ in shape))
    if l == 0:
        edge_attr, eew = ea_prev, weights["eew"]
        boff = weights["eoff"] // TE
        in_arrays = [edge_attr, gsd, eew,
                     weights["fw1"], weights["fb1"], weights["fw2"],
                     weights["fb2"], weights["lin1"]]
        in_specs = [
            pl.BlockSpec((TE, 4), lambda i, _b=boff: (i + _b, 0)),
            pl.BlockSpec((TE, 128), lambda i: (i, 0)),
            wspec((4, H)), wspec((H, F)), wspec((1, F)), wspec((F, F)),
            wspec((1, F)), wspec((H, F)),
        ]
    else:
        in_arrays = [ea_prev, gsd,
                     weights["emw"], weights["emb_b"],
                     weights["fw1"], weights["fb1"], weights["fw2"],
                     weights["fb2"], weights["lin1"]]
        in_specs = [
            pl.BlockSpec((TE, H), lambda i: (i, 0)),
            pl.BlockSpec((TE, 128), lambda i: (i, 0)),
            wspec((2 * H, H)), wspec((1, H)),
            wspec((H, F)), wspec((1, F)), wspec((F, F)), wspec((1, F)),
            wspec((H, F)),
        ]
    out_shapes = [jax.ShapeDtypeStruct((2, EH, 128), jnp.float32)]
    out_specs = [pl.BlockSpec((2, TE, 128), lambda i: (0, i, 0))]
    if want_ea:
        out_shapes.append(jax.ShapeDtypeStruct((EH, H), jnp.float32))
        out_specs.append(pl.BlockSpec((TE, H), lambda i: (i, 0)))

    res = pl.pallas_call(
        body, grid=grid, in_specs=in_specs, out_specs=out_specs,
        out_shape=out_shapes,
    )(*in_arrays)
    return res if want_ea else (res[0], None)


def _tc_node(agg2a, agg2b, h_old, lin2w, lin2b, linw, linb):
    def body(aa_ref, ab_ref, h_ref, w1_ref, b1_ref, w2_ref, b2_ref, o_ref):
        a = jnp.concatenate([aa_ref[0] + ab_ref[0], aa_ref[1] + ab_ref[1]],
                            axis=-1)
        t = _ssp(jnp.dot(a, w1_ref[...],
                         preferred_element_type=jnp.float32) + b1_ref[...])
        o = jnp.dot(t, w2_ref[...],
                    preferred_element_type=jnp.float32) + b2_ref[...]
        o_ref[...] = jnp.maximum(o, 0.0) + h_ref[...]

    wspec = lambda shape: pl.BlockSpec(shape, lambda i: tuple(0 for _ in shape))
    return pl.pallas_call(
        body,
        grid=(N // TN,),
        in_specs=[
            pl.BlockSpec((2, TN, 128), lambda i: (0, i, 0)),
            pl.BlockSpec((2, TN, 128), lambda i: (0, i, 0)),
            pl.BlockSpec((TN, H), lambda i: (i, 0)),
            wspec((F, H)), wspec((1, H)), wspec((H, H)), wspec((1, H)),
        ],
        out_specs=pl.BlockSpec((TN, H), lambda i: (i, 0)),
        out_shape=jax.ShapeDtypeStruct((N, H), jnp.float32),
    )(agg2a, agg2b, h_old, lin2w, lin2b, linw, linb)


def _tc_pool_head(h, batchf, head_w1, head_b1, head_w2, head_b2):
    nb = N // TN

    def body(h_ref, b_ref, w1_ref, b1_ref, w2_ref, b2_ref, o_ref, p_acc, c_acc):
        i = pl.program_id(0)

        @pl.when(i == 0)
        def _():
            p_acc[...] = jnp.zeros_like(p_acc)
            c_acc[...] = jnp.zeros_like(c_acc)

        gids = lax.broadcasted_iota(jnp.int32, (1, G), 1).astype(jnp.float32)
        oh = jnp.where(b_ref[...] == gids, 1.0, 0.0)
        p_acc[...] += lax.dot_general(
            oh, h_ref[...], (((0,), (0,)), ((), ())),
            preferred_element_type=jnp.float32)
        c_acc[...] += lax.dot_general(
            oh, jnp.ones((TN, H), jnp.float32), (((0,), (0,)), ((), ())),
            preferred_element_type=jnp.float32)

        @pl.when(i == nb - 1)
        def _():
            gmean = p_acc[...] / jnp.maximum(c_acc[...], 1.0)
            hh = jax.nn.gelu(
                jnp.dot(gmean, w1_ref[...],
                        preferred_element_type=jnp.float32) + b1_ref[...])
            o_ref[...] = jnp.dot(hh, w2_ref[...],
                                 preferred_element_type=jnp.float32) + b2_ref[...]

    wspec = lambda shape: pl.BlockSpec(shape, lambda i: tuple(0 for _ in shape))
    return pl.pallas_call(
        body,
        grid=(nb,),
        in_specs=[
            pl.BlockSpec((TN, H), lambda i: (i, 0)),
            pl.BlockSpec((TN, 1), lambda i: (i, 0)),
            wspec((H, 512)), wspec((1, 512)), wspec((512, 1)), wspec((1, 1)),
        ],
        out_specs=pl.BlockSpec((G, 1), lambda i: (0, 0)),
        out_shape=jax.ShapeDtypeStruct((G, 1), jnp.float32),
        scratch_shapes=[
            pltpu.VMEM((G, H), jnp.float32),
            pltpu.VMEM((G, H), jnp.float32),
        ],
    )(h, batchf, head_w1, head_b1, head_w2, head_b2)


def kernel(x, edge_index, edge_attr, batch, pos, vert_emb, pos_emb_w,
           edge_emb_w, fw1, fb1, fw2, fb2, lin1, lin2w, lin2b, linw, linb,
           emw, emb_b, head_w1, head_b1, head_w2, head_b2):
    x = x.astype(jnp.int32)
    src = edge_index[0].astype(jnp.int32)
    dst = edge_index[1].astype(jnp.int32)
    E2 = E // 2
    srcs = (src[:E2], src[E2:])
    dsts = (dst[:E2], dst[E2:])
    zrows = jnp.zeros((N // NS, 128), jnp.float32)
    batchf = batch.astype(jnp.float32).reshape(N, 1)

    npad = 10240
    xpad = jnp.concatenate([x, jnp.full((npad - N,), 300, jnp.int32)])
    vrows = _sc_gather(vert_emb, [xpad])
    h = _tc_node_init(vrows, pos, pos_emb_w)

    ea = (edge_attr, edge_attr)
    gsd = (_sc_gather(h, [srcs[0]]), _sc_gather(h, [srcs[1]]))
    for l in range(3):
        weights = dict(eew=edge_emb_w, fw1=fw1[l], fb1=fb1[l].reshape(1, F),
                       fw2=fw2[l], fb2=fb2[l].reshape(1, F), lin1=lin1[l])
        if l > 0:
            weights["emw"] = emw[l - 1]
            weights["emb_b"] = emb_b[l - 1].reshape(1, H)
        m2a, eaa = _tc_edge(l, ea[0], gsd[0], dict(weights, eoff=0))
        agg2a = _sc_scatter_add(m2a, dsts[0], zrows)
        m2b, eab = _tc_edge(l, ea[1], gsd[1], dict(weights, eoff=E2))
        agg2b = _sc_scatter_add(m2b, dsts[1], zrows)
        ea = (eaa, eab)
        h = _tc_node(agg2a, agg2b, h, lin2w[l], lin2b[l].reshape(1, H),
                     linw[l], linb[l].reshape(1, H))
        if l < 2:
            gsd = (_sc_gather(h, [srcs[0], dsts[0]]),
                   _sc_gather(h, [srcs[1], dsts[1]]))

    return _tc_pool_head(h, batchf, head_w1, head_b1.reshape(1, 512),
                         head_w2, head_b2.reshape(1, 1))


# speedup vs baseline: 1.4397x; 1.0113x over previous
import functools
import math

import jax
import jax.numpy as jnp
from jax import lax
from jax.experimental import pallas as pl
from jax.experimental.pallas import tpu as pltpu
from jax.experimental.pallas import tpu_sc as plsc

N = 10000
E = 160000
H = 64
F = 256
G = 64

NC = 2
NS = 16
NW = NC * NS

_LOG2 = math.log(2.0)

TE = 5000
TN = 1000


def _ssp(v):
    return jnp.maximum(v, 0.0) + jnp.log1p(jnp.exp(-jnp.abs(v))) - _LOG2


def _sc_gather(table, idxs):
    K = len(idxs)
    M = idxs[0].shape[0]
    D = table.shape[1]
    pw_exact = M // NW
    pad = max((w * pw_exact) % 8 for w in range(NW))
    pw = pw_exact + pad
    for w in range(NW):
        assert w * pw_exact - (w * pw_exact) % 8 + pw <= M
    CH = 128
    nfull, tail = divmod(pw, CH)
    mesh = plsc.VectorSubcoreMesh(core_axis_name="c", subcore_axis_name="s")

    def body(table_hbm, *refs):
        idx_hbms = refs[:K]
        out_hbm = refs[K]
        sc = refs[K + 1:]
        idx_vs = sc[:K]
        rbufs = sc[K:K + 2 * K]
        rtails = sc[3 * K:3 * K + K]
        gsems = sc[4 * K:4 * K + 2 * K]
        wsems = sc[6 * K:6 * K + 2 * K]
        stail = sc[8 * K]
        wid = lax.axis_index("s") * NC + lax.axis_index("c")
        b = wid * pw_exact
        base0 = pl.multiple_of(b - lax.rem(b, 8), 8)
        for k in range(K):
            pltpu.sync_copy(idx_hbms[k].at[pl.ds(base0, pw)], idx_vs[k])
        gds, wds = {}, {}
        for j in range(nfull):
            b = j % 2
            for k in range(K):
                if j >= 2:
                    wds[(j - 2, k)].wait()
                gds[(j, k)] = pltpu.async_copy(
                    table_hbm.at[idx_vs[k].at[pl.ds(j * CH, CH)]],
                    rbufs[2 * k + b], gsems[2 * k + b])
            if j >= 1:
                pb = (j - 1) % 2
                for k in range(K):
                    gds[(j - 1, k)].wait()
                    wds[(j - 1, k)] = pltpu.async_copy(
                        rbufs[2 * k + pb],
                        out_hbm.at[pl.ds(base0 + (j - 1) * CH, CH),
                                   pl.ds(64 * k, D)],
                        wsems[2 * k + pb])
        if nfull >= 1:
            j = nfull - 1
            b = j % 2
            for k in range(K):
                gds[(j, k)].wait()
                wds[(j, k)] = pltpu.async_copy(
                    rbufs[2 * k + b],
                    out_hbm.at[pl.ds(base0 + j * CH, CH), pl.ds(64 * k, D)],
                    wsems[2 * k + b])
        for j in (nfull - 2, nfull - 1):
            if j >= 0:
                for k in range(K):
                    wds[(j, k)].wait()
        if tail:
            base = base0 + nfull * CH
            for k in range(K):
                pltpu.async_copy(
                    table_hbm.at[idx_vs[k].at[pl.ds(nfull * CH, tail)]],
                    rtails[k], stail).wait()
                pltpu.async_copy(
                    rtails[k],
                    out_hbm.at[pl.ds(base, tail), pl.ds(64 * k, D)],
                    stail).wait()

    scratch = (
        [pltpu.VMEM((pw,), jnp.int32)] * K
        + [pltpu.VMEM((CH, D), jnp.float32)] * (2 * K)
        + [pltpu.VMEM((max(tail, 8), D), jnp.float32)] * K
        + [pltpu.SemaphoreType.DMA] * (4 * K + 1)
    )
    k = pl.kernel(
        body,
        out_type=jax.ShapeDtypeStruct((M, 128), jnp.float32),
        mesh=mesh,
        scratch_types=scratch,
        compiler_params=pltpu.CompilerParams(use_tc_tiling_on_sc=False),
    )
    return k(table, *idxs)


def _sc_scatter_add(m2, dst, zrows):
    EH = dst.shape[0]
    CH = 125
    nch = EH // (NS * CH)
    nps = N // NS
    mesh = plsc.VectorSubcoreMesh(core_axis_name="c", subcore_axis_name="s")

    def body(m_hbm, dst_hbm, z_hbm, agg_hbm, didx_v, r0, r1, agg_s,
             m0, m1, s0, s1):
        c = lax.axis_index("c")
        s = lax.axis_index("s")
        pltpu.sync_copy(z_hbm, agg_s.at[pl.ds(s * nps, nps)])
        pltpu.sync_copy(dst_hbm.at[pl.ds(s * nch, nch)], didx_v)
        plsc.subcore_barrier()
        bufs = [(r0, m0, s0), (r1, m1, s1)]
        mds, sds = {}, {}
        for j in range(nch):
            rows, msem, ssem = bufs[j % 2]
            if j >= 2:
                sds[j - 2].wait()
            mds[j] = pltpu.async_copy(
                m_hbm.at[c, pl.ds((s * nch + j) * CH, CH), :], rows, msem)
            if j >= 1:
                prows, _, pssem = bufs[(j - 1) % 2]
                mds[j - 1].wait()
                sds[j - 1] = pltpu.async_copy(
                    prows, agg_s.at[didx_v.at[j - 1]], pssem, add=True)
        j = nch - 1
        rows, _, ssem = bufs[j % 2]
        mds[j].wait()
        sds[j] = pltpu.async_copy(rows, agg_s.at[didx_v.at[j]], ssem, add=True)
        sds[nch - 2].wait()
        sds[nch - 1].wait()
        plsc.subcore_barrier()
        pltpu.sync_copy(agg_s.at[pl.ds(s * nps, nps)],
                        agg_hbm.at[c, pl.ds(s * nps, nps), :])

    k = pl.kernel(
        body,
        out_type=jax.ShapeDtypeStruct((2, N, 128), jnp.float32),
        mesh=mesh,
        scratch_types=[
            pltpu.VMEM((nch, CH), jnp.int32),
            pltpu.VMEM((CH, 128), jnp.float32),
            pltpu.VMEM((CH, 128), jnp.float32),
            pltpu.VMEM_SHARED((N, 128), jnp.float32),
            pltpu.SemaphoreType.DMA,
            pltpu.SemaphoreType.DMA,
            pltpu.SemaphoreType.DMA,
            pltpu.SemaphoreType.DMA,
        ],
        compiler_params=pltpu.CompilerParams(use_tc_tiling_on_sc=False),
    )
    return k(m2, dst.reshape(EH // CH, CH), zrows)


def _tc_node_init(vrows, pos, pos_emb_w):
    def body(v_ref, p_ref, w_ref, h_ref):
        pe = jnp.dot(p_ref[...], w_ref[...], preferred_element_type=jnp.float32)
        h_ref[...] = jnp.concatenate([v_ref[:, :16], pe], axis=-1)

    return pl.pallas_call(
        body,
        grid=(N // TN,),
        in_specs=[
            pl.BlockSpec((TN, 128), lambda i: (i, 0)),
            pl.BlockSpec((TN, 3), lambda i: (i, 0)),
            pl.BlockSpec((3, 48), lambda i: (0, 0)),
        ],
        out_specs=pl.BlockSpec((TN, H), lambda i: (i, 0)),
        out_shape=jax.ShapeDtypeStruct((N, H), jnp.float32),
    )(vrows, pos, pos_emb_w)


def _tc_edge(l, ea_prev, gsd, weights):
    want_ea = l < 2
    EH = gsd.shape[0]
    grid = (EH // TE,)

    def body(*refs):
        if l == 0:
            (eat_ref, gs_ref, eew_ref, fw1_ref, fb1_ref, fw2_ref, fb2_ref,
             lin1_ref, m_ref, *rest) = refs
            ea = jnp.dot(eat_ref[...], eew_ref[...],
                         preferred_element_type=jnp.float32)
            gs = gs_ref[:, :H]
        else:
            (eap_ref, gsd_ref, emw_ref, emb_ref, fw1_ref, fb1_ref,
             fw2_ref, fb2_ref, lin1_ref, m_ref, *rest) = refs
            eap = eap_ref[...]
            gs = gsd_ref[:, :H]
            hs = gs + gsd_ref[:, H:]
            e_in = jnp.concatenate(
                [jnp.maximum(eap, 0.0), jnp.maximum(hs, 0.0)], axis=-1)
            ea = jnp.tanh(
                jnp.dot(e_in, emw_ref[...],
                        preferred_element_type=jnp.float32) + emb_ref[...]
            ) + eap
        t = _ssp(jnp.dot(ea, fw1_ref[...],
                         preferred_element_type=jnp.float32) + fb1_ref[...])
        W = jnp.dot(t, fw2_ref[...],
                    preferred_element_type=jnp.float32) + fb2_ref[...]
        m = jnp.dot(gs, lin1_ref[...],
                    preferred_element_type=jnp.float32) * W
        m_ref[0] = m[:, :128]
        m_ref[1] = m[:, 128:]
        if want_ea:
            rest[0][...] = ea

    wspec = lambda shape: pl.BlockSpec(shape, lambda i: tuple(0 for _ in shape))
    if l == 0:
        edge_attr, eew = ea_prev, weights["eew"]
        boff = weights["eoff"] // TE
        in_arrays = [edge_attr, gsd, eew,
                     weights["fw1"], weights["fb1"], weights["fw2"],
                     weights["fb2"], weights["lin1"]]
        in_specs = [
            pl.BlockSpec((TE, 4), lambda i, _b=boff: (i + _b, 0)),
            pl.BlockSpec((TE, 128), lambda i: (i, 0)),
            wspec((4, H)), wspec((H, F)), wspec((1, F)), wspec((F, F)),
            wspec((1, F)), wspec((H, F)),
        ]
    else:
        in_arrays = [ea_prev, gsd,
                     weights["emw"], weights["emb_b"],
                     weights["fw1"], weights["fb1"], weights["fw2"],
                     weights["fb2"], weights["lin1"]]
        in_specs = [
            pl.BlockSpec((TE, H), lambda i: (i, 0)),
            pl.BlockSpec((TE, 128), lambda i: (i, 0)),
            wspec((2 * H, H)), wspec((1, H)),
            wspec((H, F)), wspec((1, F)), wspec((F, F)), wspec((1, F)),
            wspec((H, F)),
        ]
    out_shapes = [jax.ShapeDtypeStruct((2, EH, 128), jnp.float32)]
    out_specs = [pl.BlockSpec((2, TE, 128), lambda i: (0, i, 0))]
    if want_ea:
        out_shapes.append(jax.ShapeDtypeStruct((EH, H), jnp.float32))
        out_specs.append(pl.BlockSpec((TE, H), lambda i: (i, 0)))

    res = pl.pallas_call(
        body, grid=grid, in_specs=in_specs, out_specs=out_specs,
        out_shape=out_shapes,
    )(*in_arrays)
    return res if want_ea else (res[0], None)


def _tc_node(agg2a, agg2b, h_old, lin2w, lin2b, linw, linb):
    def body(aa_ref, ab_ref, h_ref, w1_ref, b1_ref, w2_ref, b2_ref, o_ref):
        a = jnp.concatenate([aa_ref[0] + ab_ref[0], aa_ref[1] + ab_ref[1]],
                            axis=-1)
        t = _ssp(jnp.dot(a, w1_ref[...],
                         preferred_element_type=jnp.float32) + b1_ref[...])
        o = jnp.dot(t, w2_ref[...],
                    preferred_element_type=jnp.float32) + b2_ref[...]
        o_ref[...] = jnp.maximum(o, 0.0) + h_ref[...]

    wspec = lambda shape: pl.BlockSpec(shape, lambda i: tuple(0 for _ in shape))
    return pl.pallas_call(
        body,
        grid=(N // TN,),
        in_specs=[
            pl.BlockSpec((2, TN, 128), lambda i: (0, i, 0)),
            pl.BlockSpec((2, TN, 128), lambda i: (0, i, 0)),
            pl.BlockSpec((TN, H), lambda i: (i, 0)),
            wspec((F, H)), wspec((1, H)), wspec((H, H)), wspec((1, H)),
        ],
        out_specs=pl.BlockSpec((TN, H), lambda i: (i, 0)),
        out_shape=jax.ShapeDtypeStruct((N, H), jnp.float32),
    )(agg2a, agg2b, h_old, lin2w, lin2b, linw, linb)


def _tc_pool_head(h, batchf, head_w1, head_b1, head_w2, head_b2):
    nb = N // TN

    def body(h_ref, b_ref, w1_ref, b1_ref, w2_ref, b2_ref, o_ref, p_acc, c_acc):
        i = pl.program_id(0)

        @pl.when(i == 0)
        def _():
            p_acc[...] = jnp.zeros_like(p_acc)
            c_acc[...] = jnp.zeros_like(c_acc)

        gids = lax.broadcasted_iota(jnp.int32, (1, G), 1).astype(jnp.float32)
        oh = jnp.where(b_ref[...] == gids, 1.0, 0.0)
        p_acc[...] += lax.dot_general(
            oh, h_ref[...], (((0,), (0,)), ((), ())),
            preferred_element_type=jnp.float32)
        c_acc[...] += lax.dot_general(
            oh, jnp.ones((TN, H), jnp.float32), (((0,), (0,)), ((), ())),
            preferred_element_type=jnp.float32)

        @pl.when(i == nb - 1)
        def _():
            gmean = p_acc[...] / jnp.maximum(c_acc[...], 1.0)
            hh = jax.nn.gelu(
                jnp.dot(gmean, w1_ref[...],
                        preferred_element_type=jnp.float32) + b1_ref[...])
            o_ref[...] = jnp.dot(hh, w2_ref[...],
                                 preferred_element_type=jnp.float32) + b2_ref[...]

    wspec = lambda shape: pl.BlockSpec(shape, lambda i: tuple(0 for _ in shape))
    return pl.pallas_call(
        body,
        grid=(nb,),
        in_specs=[
            pl.BlockSpec((TN, H), lambda i: (i, 0)),
            pl.BlockSpec((TN, 1), lambda i: (i, 0)),
            wspec((H, 512)), wspec((1, 512)), wspec((512, 1)), wspec((1, 1)),
        ],
        out_specs=pl.BlockSpec((G, 1), lambda i: (0, 0)),
        out_shape=jax.ShapeDtypeStruct((G, 1), jnp.float32),
        scratch_shapes=[
            pltpu.VMEM((G, H), jnp.float32),
            pltpu.VMEM((G, H), jnp.float32),
        ],
    )(h, batchf, head_w1, head_b1, head_w2, head_b2)


def kernel(x, edge_index, edge_attr, batch, pos, vert_emb, pos_emb_w,
           edge_emb_w, fw1, fb1, fw2, fb2, lin1, lin2w, lin2b, linw, linb,
           emw, emb_b, head_w1, head_b1, head_w2, head_b2):
    x = x.astype(jnp.int32)
    src = edge_index[0].astype(jnp.int32)
    dst = edge_index[1].astype(jnp.int32)
    E2 = E // 2
    srcs = (src[:E2], src[E2:])
    dsts = (dst[:E2], dst[E2:])
    zrows = jnp.zeros((N // NS, 128), jnp.float32)
    batchf = batch.astype(jnp.float32).reshape(N, 1)

    npad = 10240
    xpad = jnp.concatenate([x, jnp.full((npad - N,), 300, jnp.int32)])
    vrows = _sc_gather(vert_emb, [xpad])
    h = _tc_node_init(vrows, pos, pos_emb_w)

    ea = (edge_attr, edge_attr)
    gsd = (_sc_gather(h, [srcs[0]]), _sc_gather(h, [srcs[1]]))
    for l in range(3):
        weights = dict(eew=edge_emb_w, fw1=fw1[l], fb1=fb1[l].reshape(1, F),
                       fw2=fw2[l], fb2=fb2[l].reshape(1, F), lin1=lin1[l])
        if l > 0:
            weights["emw"] = emw[l - 1]
            weights["emb_b"] = emb_b[l - 1].reshape(1, H)
        m2a, eaa = _tc_edge(l, ea[0], gsd[0], dict(weights, eoff=0))
        agg2a = _sc_scatter_add(m2a, dsts[0], zrows)
        m2b, eab = _tc_edge(l, ea[1], gsd[1], dict(weights, eoff=E2))
        agg2b = _sc_scatter_add(m2b, dsts[1], zrows)
        ea = (eaa, eab)
        h = _tc_node(agg2a, agg2b, h, lin2w[l], lin2b[l].reshape(1, H),
                     linw[l], linb[l].reshape(1, H))
        if l < 2:
            gsd = (_sc_gather(h, [srcs[0], dsts[0]]),
                   _sc_gather(h, [srcs[1], dsts[1]]))

    return _tc_pool_head(h, batchf, head_w1, head_b1.reshape(1, 512),
                         head_w2, head_b2.reshape(1, 1))
